# baseline (speedup 1.0000x reference)
import functools

import jax
import jax.numpy as jnp
from jax import lax
from jax.experimental import pallas as pl
from jax.experimental.pallas import tpu as pltpu
from jax.experimental.pallas import tpu_sc as plsc

N = 10000
NPAD = 10240
E = 320000
DIN = 128
DOUT = 128
CHUNK = 128
EDGES_PER_CORE = E // 2
CPC = 1280
PAD_TAIL = CPC * CHUNK - EDGES_PER_CORE
NS = 16
CPT = CPC // NS
NROUND = 4
RCH = CPT // NROUND
ROWS_PER_TILE = NPAD // NS
RB = 128


def _mm_body(x_ref, w_ref, o_ref):
    o_ref[...] = jnp.dot(x_ref[...], w_ref[...], preferred_element_type=jnp.float32)


def _matmul(x, W):
    bm = 1000
    return pl.pallas_call(
        _mm_body,
        grid=(N // bm,),
        in_specs=[
            pl.BlockSpec((bm, DIN), lambda i: (i, 0)),
            pl.BlockSpec((DIN, DOUT), lambda i: (0, 0)),
        ],
        out_specs=pl.BlockSpec((bm, DOUT), lambda i: (i, 0)),
        out_shape=jax.ShapeDtypeStruct((N, DOUT), jnp.float32),
    )(x, W)


def _combine_body(p_ref, o_ref):
    o_ref[...] = jnp.maximum(p_ref[0] + p_ref[1], 0.0)


def _combine_relu(partials):
    bm = 1000
    return pl.pallas_call(
        _combine_body,
        grid=(N // bm,),
        in_specs=[pl.BlockSpec((2, bm, DOUT), lambda i: (0, i, 0))],
        out_specs=pl.BlockSpec((bm, DOUT), lambda i: (i, 0)),
        out_shape=jax.ShapeDtypeStruct((N, DOUT), jnp.float32),
    )(partials)


@functools.partial(
    pl.kernel,
    out_type=jax.ShapeDtypeStruct((2, NPAD, DOUT), jnp.float32),
    mesh=plsc.VectorSubcoreMesh(core_axis_name="c", subcore_axis_name="s"),
    scratch_types=[
        pltpu.VMEM((RCH, 1, CHUNK), jnp.int32),
        pltpu.VMEM((RCH, 1, CHUNK), jnp.int32),
        pltpu.VMEM((RCH, 1, CHUNK), jnp.float32),
        pltpu.VMEM((CHUNK, DOUT), jnp.float32),
        pltpu.VMEM((CHUNK, DOUT), jnp.float32),
        pltpu.VMEM_SHARED((NPAD, DOUT), jnp.float32),
        pltpu.SemaphoreType.DMA,
        pltpu.SemaphoreType.DMA,
        pltpu.SemaphoreType.DMA,
    ],
)
def _sc_aggregate(pre_hbm, src_hbm, dst_hbm, ew_hbm, out_hbm,
                  src_v, dst_v, ew_v, rows_a, rows_b,
                  acc, sem_i, sem_ga, sem_gb):
    c = lax.axis_index("c")
    s = lax.axis_index("s")
    row0 = s * ROWS_PER_TILE

    def _zero_row(r, carry):
        for j in range(DOUT // 16):
            rows_a[r, pl.ds(j * 16, 16)] = jnp.zeros((16,), jnp.float32)
        return carry

    lax.fori_loop(0, RB, _zero_row, 0)
    for b in range(ROWS_PER_TILE // RB):
        pltpu.sync_copy(rows_a.at[pl.ds(0, RB)],
                        acc.at[pl.ds(row0 + b * RB, RB)])
    plsc.subcore_barrier()

    def _gth(i, rv, sem):
        return pltpu.make_async_copy(pre_hbm.at[src_v.at[i, 0]], rv, sem)

    def _scale(i, rv):
        def body(eg, carry):
            w16 = ew_v[i, 0, pl.ds(eg * 16, 16)]
            for k in range(16):
                e = eg * 16 + k
                wk = w16[k]
                for j in range(DOUT // 16):
                    sl = pl.ds(j * 16, 16)
                    rv[e, sl] = rv[e, sl] * wk
            return carry

        lax.fori_loop(0, CHUNK // 16, body, 0)

    for r in range(NROUND):
        r0 = c * CPC + s * CPT + r * RCH
        pltpu.make_async_copy(src_hbm.at[pl.ds(r0, RCH)], src_v, sem_i).start()
        pltpu.make_async_copy(dst_hbm.at[pl.ds(r0, RCH)], dst_v, sem_i).start()
        pltpu.make_async_copy(ew_hbm.at[pl.ds(r0, RCH)], ew_v, sem_i).start()
        pltpu.make_async_copy(src_hbm.at[pl.ds(r0, RCH)], src_v, sem_i).wait()
        pltpu.make_async_copy(dst_hbm.at[pl.ds(r0, RCH)], dst_v, sem_i).wait()
        pltpu.make_async_copy(ew_hbm.at[pl.ds(r0, RCH)], ew_v, sem_i).wait()
        def _one(i, carry):
            _gth(i, rows_a, sem_ga).start()
            _gth(i, rows_a, sem_ga).wait()
            _scale(i, rows_a)
            pltpu.sync_copy(rows_a, acc.at[dst_v.at[i, 0]], add=True)
            return carry

        lax.fori_loop(0, RCH, _one, 0)
    plsc.subcore_barrier()

    pltpu.sync_copy(acc.at[pl.ds(row0, ROWS_PER_TILE)],
                    out_hbm.at[c, pl.ds(row0, ROWS_PER_TILE)])


def _pad_split(a):
    z = jnp.zeros((PAD_TAIL,), a.dtype)
    return jnp.concatenate(
        [a[:EDGES_PER_CORE], z, a[EDGES_PER_CORE:], z]
    ).reshape(2 * CPC, 1, CHUNK)


def kernel(x, edge_index, edge_weight, W):
    pre = _matmul(x, W)
    partials = _sc_aggregate(
        pre,
        _pad_split(edge_index[0]),
        _pad_split(edge_index[1]),
        _pad_split(edge_weight),
    )
    return _combine_relu(partials)
